# Initial kernel scaffold; baseline (speedup 1.0000x reference)
import jax
import jax.numpy as jnp
from jax.experimental import pallas as pl


def kernel(input_ids, token_type_ids, word_emb, pos_emb, type_emb, ln_gamma, ln_beta):
    raise NotImplementedError("write your pallas kernel here")


# speedup vs baseline: 1.4349x; 1.4349x over previous
import functools

import jax
import jax.numpy as jnp
from jax import lax
from jax.experimental import pallas as pl
from jax.experimental.pallas import tpu as pltpu
from jax.experimental.pallas import tpu_sc as plsc

HIDDEN = 768
EPS = 1e-12

NC = 2
NS = 16
NW = NC * NS
CH = 128


def _sc_gather(ids_flat, word_emb):
    tok = ids_flat.shape[0]
    tpw = tok // NW
    nch = tpw // CH
    mesh = plsc.VectorSubcoreMesh(core_axis_name="c", subcore_axis_name="s")

    @functools.partial(
        pl.kernel,
        mesh=mesh,
        out_type=jax.ShapeDtypeStruct((tok, HIDDEN), jnp.float32),
        scratch_types=[
            pltpu.VMEM((nch, CH), jnp.int32),
            pltpu.VMEM((CH, HIDDEN), jnp.float32),
            pltpu.SemaphoreType.DMA,
        ],
    )
    def k(ids_hbm, table_hbm, out_hbm, idx_v, rows_v, sem):
        wid = lax.axis_index("s") * NC + lax.axis_index("c")
        base = wid * tpw
        for c in range(nch):
            pltpu.sync_copy(ids_hbm.at[pl.ds(base + c * CH, CH)], idx_v.at[c])
            pltpu.async_copy(table_hbm.at[idx_v.at[c]], rows_v, sem).wait()
            pltpu.sync_copy(rows_v, out_hbm.at[pl.ds(base + c * CH, CH)])

    return k(ids_flat, word_emb)


def _ln_body(x_ref, pos_ref, tt_ref, type_ref, g_ref, b_ref, o_ref):
    x = x_ref[...]
    tt = tt_ref[...].astype(jnp.float32)
    t0 = type_ref[0:1, :]
    t1 = type_ref[1:2, :]
    x = x + pos_ref[...] + t0 + tt * (t1 - t0)
    mean = jnp.mean(x, axis=-1, keepdims=True)
    xc = x - mean
    var = jnp.mean(xc * xc, axis=-1, keepdims=True)
    y = xc * lax.rsqrt(var + EPS)
    o_ref[...] = y * g_ref[...] + b_ref[...]


def _tc_ln(gathered, pos_emb, tt_flat, type_emb, gamma, beta):
    tok = gathered.shape[0]
    seq = pos_emb.shape[0]
    br = 256
    grid = (tok // br,)
    pos_blocks = seq // br
    return pl.pallas_call(
        _ln_body,
        grid=grid,
        in_specs=[
            pl.BlockSpec((br, HIDDEN), lambda g: (g, 0)),
            pl.BlockSpec((br, HIDDEN), lambda g: (g % pos_blocks, 0)),
            pl.BlockSpec((br, 1), lambda g: (g, 0)),
            pl.BlockSpec((2, HIDDEN), lambda g: (0, 0)),
            pl.BlockSpec((1, HIDDEN), lambda g: (0, 0)),
            pl.BlockSpec((1, HIDDEN), lambda g: (0, 0)),
        ],
        out_specs=pl.BlockSpec((br, HIDDEN), lambda g: (g, 0)),
        out_shape=jax.ShapeDtypeStruct((tok, HIDDEN), jnp.float32),
    )(gathered, pos_emb, tt_flat, type_emb, gamma, beta)


def kernel(input_ids, token_type_ids, word_emb, pos_emb, type_emb, ln_gamma, ln_beta):
    b, s = input_ids.shape
    ids_flat = input_ids.reshape(-1).astype(jnp.int32)
    tt_flat = token_type_ids.reshape(-1, 1).astype(jnp.int32)
    gathered = _sc_gather(ids_flat, word_emb)
    pos_slice = pos_emb[:s]
    out = _tc_ln(gathered, pos_slice, tt_flat, type_emb,
                 ln_gamma.reshape(1, HIDDEN), ln_beta.reshape(1, HIDDEN))
    return out.reshape(b, s, HIDDEN)
